# Initial kernel scaffold; baseline (speedup 1.0000x reference)
import jax
import jax.numpy as jnp
from jax.experimental import pallas as pl


def kernel(x, edge_index, W1, b1, W2, b2, Wfc, bfc):
    raise NotImplementedError("write your pallas kernel here")


# speedup vs baseline: 54.6723x; 54.6723x over previous
import functools

import jax
import jax.numpy as jnp
from jax import lax
from jax.experimental import pallas as pl
from jax.experimental.pallas import tpu as pltpu
from jax.experimental.pallas import tpu_sc as plsc

N = 100100
E = 3203200
D = 3
NODES_PER_GRAPH = 286
NG_GRAPHS = 350

NC = 2
NS = 16
NW = NC * NS
LANES = 16

NSP = 100352
NPT = NSP // NS
BATCH = 128
K = 8
NB = 25088
NBT = NB // NW
NGRP = NBT // K
EPAD = NB * BATCH

_mesh = plsc.VectorSubcoreMesh(
    core_axis_name="c", subcore_axis_name="s", num_cores=NC, num_subcores=NS)


@functools.partial(
    pl.kernel,
    out_type=jax.ShapeDtypeStruct((NC, NSP), jnp.float32),
    mesh=_mesh,
    scratch_types=[
        pltpu.VMEM((K, BATCH), jnp.int32),
        pltpu.VMEM((K, BATCH), jnp.int32),
        pltpu.VMEM((BATCH,), jnp.float32),
        pltpu.VMEM_SHARED((NSP,), jnp.float32),
        pltpu.SemaphoreType.DMA,
        pltpu.SemaphoreType.DMA,
        pltpu.SemaphoreType.DMA,
    ],
)
def _deg_pass(ei_ref, zrow_ref, ones_ref, out_ref,
              idx0, idx1, onesv, degsh, semL0, semL1, semS):
    cid = lax.axis_index("c")
    sid = lax.axis_index("s")
    wid = cid * NS + sid
    row0 = sid * NPT

    pltpu.sync_copy(zrow_ref, degsh.at[pl.ds(row0, NPT)])
    pltpu.sync_copy(ones_ref, onesv)
    plsc.subcore_barrier()

    base = wid * NBT

    def _load(g, buf, sem):
        return pltpu.async_copy(ei_ref.at[0, pl.ds(base + g * K, K)], buf, sem)

    def _issue_scatter(buf):
        for j in range(K):
            pltpu.async_copy(onesv, degsh.at[buf.at[j]], semS, add=True)

    def _wait_scatter(buf):
        for j in range(K):
            pltpu.make_async_copy(onesv, degsh.at[buf.at[j]], semS).wait()

    pltpu.sync_copy(ei_ref.at[0, pl.ds(base, K)], idx0)

    def half(g, ib, ib_n, sem_n, first, last):
        @pl.when(jnp.logical_not(first))
        def _():
            _wait_scatter(ib_n)

        @pl.when(jnp.logical_not(last))
        def _():
            _load(g + 1, ib_n, sem_n)

        _issue_scatter(ib)

        @pl.when(jnp.logical_not(last))
        def _():
            pltpu.make_async_copy(ei_ref.at[0, pl.ds(base, K)], ib_n,
                                  sem_n).wait()

    def body(i, carry):
        g0 = 2 * i
        half(g0, idx0, idx1, semL1, i == 0, jnp.bool_(False))
        half(g0 + 1, idx1, idx0, semL0, jnp.bool_(False), i == NGRP // 2 - 1)
        return carry

    lax.fori_loop(0, NGRP // 2, body, 0)
    _wait_scatter(idx1)

    plsc.subcore_barrier()
    pltpu.sync_copy(degsh.at[pl.ds(row0, NPT)],
                    out_ref.at[cid, pl.ds(row0, NPT)])


@functools.partial(
    pl.kernel,
    out_type=jax.ShapeDtypeStruct((NC, 8, NSP), jnp.float32),
    mesh=_mesh,
    compiler_params=pltpu.CompilerParams(use_tc_tiling_on_sc=False, needs_layout_passes=False),
    scratch_types=[
        pltpu.VMEM((2, K, BATCH), jnp.int32),
        pltpu.VMEM((2, K, BATCH), jnp.int32),
        pltpu.VMEM((K, BATCH, 8), jnp.float32),
        pltpu.VMEM((K, BATCH, 8), jnp.float32),
        pltpu.VMEM((224, 8), jnp.float32),
        pltpu.VMEM((8 * 224,), jnp.float32),
        pltpu.VMEM_SHARED((NSP, 8), jnp.float32),
        pltpu.VMEM_SHARED((NSP, 8), jnp.float32),
        pltpu.SemaphoreType.DMA,
        pltpu.SemaphoreType.DMA,
        pltpu.SemaphoreType.DMA,
        pltpu.SemaphoreType.DMA,
    ],
)
def _edge_pass(ei_ref, table_ref, z4_ref, out_ref,
               idx0, idx1, rows0, rows1, tb, tout, tblsh, aggsh, semL0, semL1,
               semG, semS):
    cid = lax.axis_index("c")
    sid = lax.axis_index("s")
    wid = cid * NS + sid
    row0 = sid * NPT

    pltpu.sync_copy(z4_ref, aggsh.at[pl.ds(row0, NPT)])
    pltpu.sync_copy(table_ref.at[pl.ds(row0, NPT)], tblsh.at[pl.ds(row0, NPT)])
    plsc.subcore_barrier()

    base = wid * NBT

    def _load(g, buf, sem):
        pltpu.async_copy(ei_ref.at[:, pl.ds(base + g * K, K)], buf, sem)

    def _wait_load(buf, sem):
        pltpu.make_async_copy(ei_ref.at[:, pl.ds(base, K)], buf, sem).wait()

    def _issue_gathers(ib, rb):
        for j in range(K):
            pltpu.async_copy(tblsh.at[ib.at[0, j]], rb.at[j], semG)

    def _wait_gathers(ib, rb):
        for j in range(K):
            pltpu.make_async_copy(tblsh.at[ib.at[0, j]], rb.at[j],
                                  semG).wait()

    def _issue_scatters(ib, rb):
        for j in range(K):
            pltpu.async_copy(rb.at[j], aggsh.at[ib.at[1, j]], semS, add=True)

    def _wait_scatters(ib, rb):
        for j in range(K):
            pltpu.make_async_copy(rb.at[j], aggsh.at[ib.at[1, j]],
                                  semS).wait()

    pltpu.sync_copy(ei_ref.at[:, pl.ds(base, K)], idx0)
    _issue_gathers(idx0, rows0)

    def half(i, g, ib, rb, ib_n, rb_n, first, last):
        @pl.when(jnp.logical_not(first))
        def _():
            _wait_scatters(ib_n, rb_n)

        @pl.when(jnp.logical_not(last))
        def _():
            _load(g + 1, ib_n, semL0 if ib_n is idx0 else semL1)

        _wait_gathers(ib, rb)
        _issue_scatters(ib, rb)

        @pl.when(jnp.logical_not(last))
        def _():
            _wait_load(ib_n, semL0 if ib_n is idx0 else semL1)
            _issue_gathers(ib_n, rb_n)

    def body(i, carry):
        g0 = 2 * i
        half(i, g0, idx0, rows0, idx1, rows1, i == 0, jnp.bool_(False))
        half(i, g0 + 1, idx1, rows1, idx0, rows0, jnp.bool_(False),
             i == NGRP // 2 - 1)
        return carry

    lax.fori_loop(0, NGRP // 2, body, 0)
    _wait_scatters(idx1, rows1)

    plsc.subcore_barrier()

    i16 = lax.iota(jnp.int32, 16)
    for sub in range(28):
        r0 = row0 + sub * 224
        pltpu.sync_copy(aggsh.at[pl.ds(r0, 224)], tb)

        def tbody(g, carry):
            ir = g * 16 + i16
            for c in range(8):
                ic = jnp.full((16,), c, jnp.int32)
                val = plsc.load_gather(tb, [ir, ic])
                tout[pl.ds(c * 224 + g * 16, 16)] = val
            return carry

        lax.fori_loop(0, 14, tbody, 0)
        for c in range(8):
            pltpu.sync_copy(tout.at[pl.ds(c * 224, 224)],
                            out_ref.at[cid, c, pl.ds(r0, 224)])


_BC = 12544


def _tca_body(degp_ref, xt_ref, h0n_ref, dsrc_ref):
    g = pl.program_id(0)
    deg = degp_ref[0:1, :] + degp_ref[1:2, :]
    dsrc = lax.rsqrt(jnp.maximum(deg, 1.0))
    ids = g * _BC + lax.broadcasted_iota(jnp.int32, (1, _BC), 1)
    dsrc_ref[...] = dsrc
    h0n_ref[0:3, :] = xt_ref[...] * dsrc
    h0n_ref[3:4, :] = jnp.where(ids < N, 1.0, 0.0)
    h0n_ref[4:8, :] = jnp.zeros((4, _BC), jnp.float32)


def _tcb_body(aggp_ref, dsrc_ref, par_ref, h1n_ref, din_ref):
    g = pl.program_id(0)
    agg = aggp_ref[0] + aggp_ref[1]
    din = lax.rsqrt(jnp.maximum(agg[3:4, :], 1.0))
    ids = g * _BC + lax.broadcasted_iota(jnp.int32, (1, _BC), 1)
    valid = ids < N
    a = [agg[c:c + 1, :] * din for c in range(3)]
    dsrc = dsrc_ref[...]
    for c in range(3):
        h = a[0] * par_ref[0, c] + a[1] * par_ref[0, 3 + c] \
            + a[2] * par_ref[0, 6 + c] + par_ref[0, 9 + c]
        h = jnp.maximum(h, 0.0)
        h1n_ref[c:c + 1, :] = jnp.where(valid, h * dsrc, 0.0)
    h1n_ref[3:8, :] = jnp.zeros((5, _BC), jnp.float32)
    din_ref[...] = din


def _tcc_body(aggp_ref, din_ref, par_ref, h2_ref):
    agg = aggp_ref[0] + aggp_ref[1]
    din = din_ref[...]
    a = [agg[c:c + 1, :] * din for c in range(3)]
    for c in range(3):
        h = a[0] * par_ref[0, c] + a[1] * par_ref[0, 3 + c] \
            + a[2] * par_ref[0, 6 + c] + par_ref[0, 9 + c]
        h2_ref[c:c + 1, :] = jnp.maximum(h, 0.0)


def _tcd_body(h_ref, w_ref, par_ref, out_ref):
    z = jnp.dot(h_ref[...], w_ref[...], preferred_element_type=jnp.float32)
    out_ref[...] = jax.nn.sigmoid(z + par_ref[0, 0])


def _tca(degp, xt):
    return pl.pallas_call(
        _tca_body,
        grid=(NSP // _BC,),
        in_specs=[
            pl.BlockSpec((NC, _BC), lambda i: (0, i)),
            pl.BlockSpec((3, _BC), lambda i: (0, i)),
        ],
        out_specs=[
            pl.BlockSpec((8, _BC), lambda i: (0, i)),
            pl.BlockSpec((1, _BC), lambda i: (0, i)),
        ],
        out_shape=[
            jax.ShapeDtypeStruct((8, NSP), jnp.float32),
            jax.ShapeDtypeStruct((1, NSP), jnp.float32),
        ],
    )(degp, xt)


def _tcb(aggp_t, dsrc, par):
    return pl.pallas_call(
        _tcb_body,
        grid=(NSP // _BC,),
        in_specs=[
            pl.BlockSpec((NC, 8, _BC), lambda i: (0, 0, i)),
            pl.BlockSpec((1, _BC), lambda i: (0, i)),
            pl.BlockSpec((1, 128), lambda i: (0, 0)),
        ],
        out_specs=[
            pl.BlockSpec((8, _BC), lambda i: (0, i)),
            pl.BlockSpec((1, _BC), lambda i: (0, i)),
        ],
        out_shape=[
            jax.ShapeDtypeStruct((8, NSP), jnp.float32),
            jax.ShapeDtypeStruct((1, NSP), jnp.float32),
        ],
    )(aggp_t, dsrc, par)


def _tcc(aggp_t, din, par):
    return pl.pallas_call(
        _tcc_body,
        grid=(NSP // _BC,),
        in_specs=[
            pl.BlockSpec((NC, 8, _BC), lambda i: (0, 0, i)),
            pl.BlockSpec((1, _BC), lambda i: (0, i)),
            pl.BlockSpec((1, 128), lambda i: (0, 0)),
        ],
        out_specs=pl.BlockSpec((3, _BC), lambda i: (0, i)),
        out_shape=jax.ShapeDtypeStruct((3, NSP), jnp.float32),
    )(aggp_t, din, par)


def _tcd(hp, wp, par):
    return pl.pallas_call(
        _tcd_body,
        out_shape=jax.ShapeDtypeStruct((352, 128), jnp.float32),
    )(hp, wp, par)


def kernel(x, edge_index, W1, b1, W2, b2, Wfc, bfc):
    f32 = jnp.float32
    eip = jnp.pad(edge_index.astype(jnp.int32), ((0, 0), (0, EPAD - E)),
                  constant_values=N).reshape(2, NB, BATCH)
    xt = jnp.pad(x, ((0, NSP - N), (0, 0))).T
    zrow = jnp.zeros((NPT,), f32)
    z4 = jnp.zeros((NPT, 8), f32)
    ones128 = jnp.ones((BATCH,), f32)

    par1 = jnp.zeros((1, 128), f32).at[0, :9].set(W1.reshape(-1)) \
        .at[0, 9:12].set(b1)
    par2 = jnp.zeros((1, 128), f32).at[0, :9].set(W2.reshape(-1)) \
        .at[0, 9:12].set(b2)
    par3 = jnp.zeros((1, 128), f32).at[0, 0].set(bfc[0])

    degp = _deg_pass(eip, zrow, ones128)
    h0n_t, dsrc = _tca(degp, xt)
    aggp1 = _edge_pass(eip, h0n_t.T, z4)
    h1n_t, din = _tcb(aggp1, dsrc, par1)
    aggp2 = _edge_pass(eip, h1n_t.T, z4)
    h2_t = _tcc(aggp2, din, par2)

    h2 = h2_t.T[:N].reshape(NG_GRAPHS, NODES_PER_GRAPH * D)
    hp = jnp.pad(h2, ((0, 2), (0, 38)))
    wfcp = jnp.pad(Wfc, ((0, 38), (0, 127)))
    out = _tcd(hp, wfcp, par3)
    return out[:NG_GRAPHS, 0:1]
